# baseline (speedup 1.0000x reference)
import functools

import jax
import jax.numpy as jnp
from jax.experimental import pallas as pl
from jax.experimental.pallas import tpu as pltpu


def _dense_relu_kernel(x_ref, w_ref, b_ref, o_ref):
    acc = jnp.dot(x_ref[...], w_ref[...], preferred_element_type=jnp.float32)
    o_ref[...] = jnp.maximum(acc + b_ref[...], 0.0)


def _dense_relu(x2d, W, b, block_rows=1024):
    R, Cin = x2d.shape
    Cout = W.shape[1]
    br = min(block_rows, max(8, R))
    Rp = ((R + br - 1) // br) * br
    if Rp != R:
        x2d = jnp.pad(x2d, ((0, Rp - R), (0, 0)))
    out = pl.pallas_call(
        _dense_relu_kernel,
        grid=(Rp // br,),
        in_specs=[
            pl.BlockSpec((br, Cin), lambda i: (i, 0)),
            pl.BlockSpec((Cin, Cout), lambda i: (0, 0)),
            pl.BlockSpec((1, Cout), lambda i: (0, 0)),
        ],
        out_specs=pl.BlockSpec((br, Cout), lambda i: (i, 0)),
        out_shape=jax.ShapeDtypeStruct((Rp, Cout), jnp.float32),
    )(x2d, W, b.reshape(1, Cout))
    return out[:R]


def _mlp(x, Ws, bs):
    shape = x.shape
    h = x.reshape(-1, shape[-1])
    for W, b in zip(Ws, bs):
        h = _dense_relu(h, W, b)
    return h.reshape(*shape[:-1], h.shape[-1])


def _square_distance(src, dst):
    d = -2.0 * jnp.einsum('bsc,bmc->bsm', src, dst)
    d = d + jnp.sum(src ** 2, -1)[:, :, None]
    d = d + jnp.sum(dst ** 2, -1)[:, None, :]
    return d


def _index_points(points, idx):
    return jax.vmap(lambda p, i: p[i])(points, idx)


def _ball_query(radius, nsample, xyz, new_xyz):
    Np = xyz.shape[1]
    sqr = _square_distance(new_xyz, xyz)
    idx = jnp.broadcast_to(jnp.arange(Np, dtype=jnp.int32), sqr.shape)
    idx = jnp.where(sqr > radius * radius, Np, idx)
    idx = jnp.sort(idx, axis=-1)[:, :, :nsample]
    first = idx[:, :, :1]
    idx = jnp.where(idx == Np, first, idx)
    return idx


def _fps(xyz, npoint):
    Bc, Nc, _ = xyz.shape
    def step(state, _):
        distance, farthest = state
        centroid = jax.vmap(lambda p, i: p[i])(xyz, farthest)[:, None, :]
        dist = jnp.sum((xyz - centroid) ** 2, -1)
        distance = jnp.minimum(distance, dist)
        new_far = jnp.argmax(distance, -1).astype(jnp.int32)
        return (distance, new_far), farthest
    init = (jnp.full((Bc, Nc), 1e10, dtype=jnp.float32), jnp.zeros((Bc,), jnp.int32))
    _, idxs = jax.lax.scan(step, init, None, length=npoint)
    return jnp.transpose(idxs)


def _relation_encoding(xyz_t, Ws, bs):
    idx = _ball_query(0.2, 32, xyz_t, xyz_t)
    grouped = _index_points(xyz_t, idx)
    norm = grouped - xyz_t[:, :, None, :]
    feat = jnp.concatenate([norm, grouped], -1)
    h = _mlp(feat, Ws, bs)
    maxf = jnp.max(h, axis=2)
    avgf = jnp.mean(h, axis=2)
    return jnp.concatenate([maxf, avgf], -1)


def _set_abstraction(xyz_t, points_t, npoint, radius, nsample, Ws, bs, group_all):
    Bc = xyz_t.shape[0]
    if group_all:
        grouped_xyz = xyz_t[:, None]
        feat = jnp.concatenate([grouped_xyz, grouped_xyz, points_t[:, None]], -1)
        h = _mlp(feat, Ws, bs)
        new_points = jnp.max(h, axis=2)
        return jnp.zeros((Bc, 1, 3), jnp.float32), new_points, jnp.float32(0.0), xyz_t.shape[1]
    fps_idx = _fps(xyz_t, npoint)
    new_xyz = _index_points(xyz_t, fps_idx)
    idx = _ball_query(radius, nsample, xyz_t, new_xyz)
    grouped_xyz = _index_points(xyz_t, idx)
    norm = grouped_xyz - new_xyz[:, :, None, :]
    grouped_pts = _index_points(points_t, idx)
    feat = jnp.concatenate([norm, grouped_xyz, grouped_pts], -1)
    h = _mlp(feat, Ws, bs)
    maxf = jnp.max(h, 2)
    avgf = jnp.mean(h, 2)
    nm = jnp.sqrt(jnp.sum(maxf ** 2, -1)) * jnp.sqrt(jnp.sum(avgf ** 2, -1)) + 1e-8
    cos_loss = jnp.mean(jnp.sum(maxf * avgf, -1) / nm)
    new_points = jnp.concatenate([maxf, avgf], -1)
    return new_xyz, new_points, cos_loss, npoint


def _batchnorm1d(x, g, b):
    m = jnp.mean(x, 0)
    v = jnp.var(x, 0)
    return (x - m) / jnp.sqrt(v + 1e-5) * g + b


def kernel(xyz, params):
    xyz_t = jnp.transpose(xyz, (0, 2, 1))
    points = _relation_encoding(xyz_t, params['re_W'], params['re_b'])
    l1_xyz, l1_pts, c1, u1 = _set_abstraction(
        xyz_t, points, 512, 0.2, 32, params['sa1_W'], params['sa1_b'], False)
    l2_xyz, l2_pts, c2, u2 = _set_abstraction(
        l1_xyz, l1_pts, 128, 0.4, 20, params['sa2_W'], params['sa2_b'], False)
    l3_xyz, l3_pts, c3, _ = _set_abstraction(
        l2_xyz, l2_pts, 1, None, 128, params['sa3_W'], params['sa3_b'], True)
    x = l3_pts.reshape(xyz.shape[0], 1024)
    x = jax.nn.leaky_relu(
        _batchnorm1d(x @ params['fc1_W'] + params['fc1_b'], params['bn1_g'], params['bn1_b']), 0.2)
    x = jax.nn.leaky_relu(
        _batchnorm1d(x @ params['fc2_W'] + params['fc2_b'], params['bn2_g'], params['bn2_b']), 0.2)
    x = x @ params['fc3_W'] + params['fc3_b']
    cos_loss = c1 + c2 + c3
    return (x, cos_loss, jnp.asarray(u1), jnp.asarray(u2))


# speedup vs baseline: 1.2435x; 1.2435x over previous
import functools

import jax
import jax.numpy as jnp
from jax.experimental import pallas as pl
from jax.experimental.pallas import tpu as pltpu


def _dense_relu_kernel(x_ref, w_ref, b_ref, o_ref):
    acc = jnp.dot(x_ref[...], w_ref[...], preferred_element_type=jnp.float32)
    o_ref[...] = jnp.maximum(acc + b_ref[...], 0.0)


def _dense_relu(x2d, W, b, block_rows=1024):
    R, Cin = x2d.shape
    Cout = W.shape[1]
    br = min(block_rows, max(8, R))
    Rp = ((R + br - 1) // br) * br
    if Rp != R:
        x2d = jnp.pad(x2d, ((0, Rp - R), (0, 0)))
    out = pl.pallas_call(
        _dense_relu_kernel,
        grid=(Rp // br,),
        in_specs=[
            pl.BlockSpec((br, Cin), lambda i: (i, 0)),
            pl.BlockSpec((Cin, Cout), lambda i: (0, 0)),
            pl.BlockSpec((1, Cout), lambda i: (0, 0)),
        ],
        out_specs=pl.BlockSpec((br, Cout), lambda i: (i, 0)),
        out_shape=jax.ShapeDtypeStruct((Rp, Cout), jnp.float32),
    )(x2d, W, b.reshape(1, Cout))
    return out[:R]


def _mlp(x, Ws, bs):
    shape = x.shape
    h = x.reshape(-1, shape[-1])
    for W, b in zip(Ws, bs):
        h = _dense_relu(h, W, b)
    return h.reshape(*shape[:-1], h.shape[-1])


def _bq_kernel(q_ref, t_ref, o_ref, *, radius, K, N):
    q = q_ref[0]
    t = t_ref[0]
    dot = jax.lax.dot_general(q, t, (((1,), (1,)), ((), ())),
                              preferred_element_type=jnp.float32)
    qn2 = jnp.sum(q * q, axis=1, keepdims=True)
    tn2 = jnp.sum(t * t, axis=1)[None, :]
    d = -2.0 * dot + qn2 + tn2
    mask = (d <= radius * radius).astype(jnp.float32)
    ch = min(128, N)
    row_i = jax.lax.broadcasted_iota(jnp.int32, (ch, ch), 0)
    col_i = jax.lax.broadcasted_iota(jnp.int32, (ch, ch), 1)
    ut = (row_i <= col_i).astype(jnp.float32)
    chunks = []
    carry = jnp.zeros((q.shape[0], 1), jnp.float32)
    for c in range(N // ch):
        blk = mask[:, c * ch:(c + 1) * ch]
        r = jnp.dot(blk, ut, preferred_element_type=jnp.float32) + carry
        carry = r[:, ch - 1:ch]
        chunks.append(r)
    rank = jnp.concatenate(chunks, axis=1)
    cols = []
    for k in range(K):
        cols.append(jnp.sum(jnp.where(rank <= float(k), 1.0, 0.0),
                            axis=-1, keepdims=True))
    idx = jnp.concatenate(cols, axis=1)
    first = idx[:, 0:1]
    idx = jnp.where(idx >= float(N), first, idx)
    o_ref[0] = idx.astype(jnp.int32)


def _pad8(x3):
    B_, M, _ = x3.shape
    return jnp.concatenate([x3, jnp.zeros((B_, M, 5), jnp.float32)], axis=-1)


def _ball_query_pallas(radius, K, table8, query8, qb=256):
    Bc, N, _ = table8.shape
    Q = query8.shape[1]
    qb = min(qb, Q)
    kern = functools.partial(_bq_kernel, radius=radius, K=K, N=N)
    return pl.pallas_call(
        kern,
        grid=(Bc, Q // qb),
        in_specs=[
            pl.BlockSpec((1, qb, 8), lambda b, i: (b, i, 0)),
            pl.BlockSpec((1, N, 8), lambda b, i: (b, 0, 0)),
        ],
        out_specs=pl.BlockSpec((1, qb, K), lambda b, i: (b, i, 0)),
        out_shape=jax.ShapeDtypeStruct((Bc, Q, K), jnp.int32),
    )(query8, table8)


def _square_distance(src, dst):
    d = -2.0 * jnp.einsum('bsc,bmc->bsm', src, dst)
    d = d + jnp.sum(src ** 2, -1)[:, :, None]
    d = d + jnp.sum(dst ** 2, -1)[:, None, :]
    return d


def _index_points(points, idx):
    return jax.vmap(lambda p, i: p[i])(points, idx)


def _ball_query(radius, nsample, xyz, new_xyz):
    Np = xyz.shape[1]
    sqr = _square_distance(new_xyz, xyz)
    idx = jnp.broadcast_to(jnp.arange(Np, dtype=jnp.int32), sqr.shape)
    idx = jnp.where(sqr > radius * radius, Np, idx)
    idx = jnp.sort(idx, axis=-1)[:, :, :nsample]
    first = idx[:, :, :1]
    idx = jnp.where(idx == Np, first, idx)
    return idx


def _fps(xyz, npoint):
    Bc, Nc, _ = xyz.shape
    def step(state, _):
        distance, farthest = state
        centroid = jax.vmap(lambda p, i: p[i])(xyz, farthest)[:, None, :]
        dist = jnp.sum((xyz - centroid) ** 2, -1)
        distance = jnp.minimum(distance, dist)
        new_far = jnp.argmax(distance, -1).astype(jnp.int32)
        return (distance, new_far), farthest
    init = (jnp.full((Bc, Nc), 1e10, dtype=jnp.float32), jnp.zeros((Bc,), jnp.int32))
    _, idxs = jax.lax.scan(step, init, None, length=npoint)
    return jnp.transpose(idxs)


def _relation_encoding(xyz_t, Ws, bs):
    x8 = _pad8(xyz_t)
    idx = _ball_query_pallas(0.2, 32, x8, x8)
    grouped = _index_points(xyz_t, idx)
    norm = grouped - xyz_t[:, :, None, :]
    feat = jnp.concatenate([norm, grouped], -1)
    h = _mlp(feat, Ws, bs)
    maxf = jnp.max(h, axis=2)
    avgf = jnp.mean(h, axis=2)
    return jnp.concatenate([maxf, avgf], -1)


def _set_abstraction(xyz_t, points_t, npoint, radius, nsample, Ws, bs, group_all):
    Bc = xyz_t.shape[0]
    if group_all:
        grouped_xyz = xyz_t[:, None]
        feat = jnp.concatenate([grouped_xyz, grouped_xyz, points_t[:, None]], -1)
        h = _mlp(feat, Ws, bs)
        new_points = jnp.max(h, axis=2)
        return jnp.zeros((Bc, 1, 3), jnp.float32), new_points, jnp.float32(0.0), xyz_t.shape[1]
    fps_idx = _fps(xyz_t, npoint)
    new_xyz = _index_points(xyz_t, fps_idx)
    idx = _ball_query_pallas(radius, nsample, _pad8(xyz_t), _pad8(new_xyz),
                             qb=128 if npoint == 128 else 256)
    grouped_xyz = _index_points(xyz_t, idx)
    norm = grouped_xyz - new_xyz[:, :, None, :]
    grouped_pts = _index_points(points_t, idx)
    feat = jnp.concatenate([norm, grouped_xyz, grouped_pts], -1)
    h = _mlp(feat, Ws, bs)
    maxf = jnp.max(h, 2)
    avgf = jnp.mean(h, 2)
    nm = jnp.sqrt(jnp.sum(maxf ** 2, -1)) * jnp.sqrt(jnp.sum(avgf ** 2, -1)) + 1e-8
    cos_loss = jnp.mean(jnp.sum(maxf * avgf, -1) / nm)
    new_points = jnp.concatenate([maxf, avgf], -1)
    return new_xyz, new_points, cos_loss, npoint


def _batchnorm1d(x, g, b):
    m = jnp.mean(x, 0)
    v = jnp.var(x, 0)
    return (x - m) / jnp.sqrt(v + 1e-5) * g + b


def kernel(xyz, params):
    xyz_t = jnp.transpose(xyz, (0, 2, 1))
    points = _relation_encoding(xyz_t, params['re_W'], params['re_b'])
    l1_xyz, l1_pts, c1, u1 = _set_abstraction(
        xyz_t, points, 512, 0.2, 32, params['sa1_W'], params['sa1_b'], False)
    l2_xyz, l2_pts, c2, u2 = _set_abstraction(
        l1_xyz, l1_pts, 128, 0.4, 20, params['sa2_W'], params['sa2_b'], False)
    l3_xyz, l3_pts, c3, _ = _set_abstraction(
        l2_xyz, l2_pts, 1, None, 128, params['sa3_W'], params['sa3_b'], True)
    x = l3_pts.reshape(xyz.shape[0], 1024)
    x = jax.nn.leaky_relu(
        _batchnorm1d(x @ params['fc1_W'] + params['fc1_b'], params['bn1_g'], params['bn1_b']), 0.2)
    x = jax.nn.leaky_relu(
        _batchnorm1d(x @ params['fc2_W'] + params['fc2_b'], params['bn2_g'], params['bn2_b']), 0.2)
    x = x @ params['fc3_W'] + params['fc3_b']
    cos_loss = c1 + c2 + c3
    return (x, cos_loss, jnp.asarray(u1), jnp.asarray(u2))


# speedup vs baseline: 3.4889x; 2.8056x over previous
import functools

import jax
import jax.numpy as jnp
from jax.experimental import pallas as pl
from jax.experimental.pallas import tpu as pltpu

B = 8


def _dense_relu_kernel(x_ref, w_ref, b_ref, o_ref):
    acc = jnp.dot(x_ref[...], w_ref[...], preferred_element_type=jnp.float32)
    o_ref[...] = jnp.maximum(acc + b_ref[...], 0.0)


def _dense_relu(x2d, W, b, block_rows=1024):
    R, Cin = x2d.shape
    Cout = W.shape[1]
    br = min(block_rows, max(8, R))
    Rp = ((R + br - 1) // br) * br
    if Rp != R:
        x2d = jnp.pad(x2d, ((0, Rp - R), (0, 0)))
    out = pl.pallas_call(
        _dense_relu_kernel,
        grid=(Rp // br,),
        in_specs=[
            pl.BlockSpec((br, Cin), lambda i: (i, 0)),
            pl.BlockSpec((Cin, Cout), lambda i: (0, 0)),
            pl.BlockSpec((1, Cout), lambda i: (0, 0)),
        ],
        out_specs=pl.BlockSpec((br, Cout), lambda i: (i, 0)),
        out_shape=jax.ShapeDtypeStruct((Rp, Cout), jnp.float32),
    )(x2d, W, b.reshape(1, Cout))
    return out[:R]


def _bq_kernel(q_ref, t_ref, o_ref, *, radius, K, Kp, N):
    q = q_ref[0]
    t = t_ref[0]
    b = pl.program_id(0)
    dot = jax.lax.dot_general(q, t, (((1,), (1,)), ((), ())),
                              preferred_element_type=jnp.float32)
    qn2 = jnp.sum(q * q, axis=1, keepdims=True)
    tn2 = jnp.sum(t * t, axis=1)[None, :]
    d = -2.0 * dot + qn2 + tn2
    mask = (d <= radius * radius).astype(jnp.float32)
    ch = min(128, N)
    row_i = jax.lax.broadcasted_iota(jnp.int32, (ch, ch), 0)
    col_i = jax.lax.broadcasted_iota(jnp.int32, (ch, ch), 1)
    ut = (row_i <= col_i).astype(jnp.float32)
    chunks = []
    carry = jnp.zeros((q.shape[0], 1), jnp.float32)
    for c in range(N // ch):
        blk = mask[:, c * ch:(c + 1) * ch]
        r = jnp.dot(blk, ut, preferred_element_type=jnp.float32) + carry
        carry = r[:, ch - 1:ch]
        chunks.append(r)
    rank = jnp.concatenate(chunks, axis=1)
    cols = []
    for k in range(K):
        cols.append(jnp.sum(jnp.where(rank <= float(k), 1.0, 0.0),
                            axis=-1, keepdims=True))
    idx = jnp.concatenate(cols, axis=1)
    first = idx[:, 0:1]
    idx = jnp.where(idx >= float(N), first, idx)
    if Kp > K:
        idx = jnp.concatenate(
            [idx, jnp.zeros((idx.shape[0], Kp - K), jnp.float32)], axis=1)
    o_ref[0] = idx.astype(jnp.int32) + b * N


def _pad8(x3):
    B_, M, _ = x3.shape
    return jnp.concatenate([x3, jnp.zeros((B_, M, 5), jnp.float32)], axis=-1)


def _ball_query_pallas(radius, K, Kp, table8, query8, qb=256):
    Bc, N, _ = table8.shape
    Q = query8.shape[1]
    qb = min(qb, Q)
    kern = functools.partial(_bq_kernel, radius=radius, K=K, Kp=Kp, N=N)
    return pl.pallas_call(
        kern,
        grid=(Bc, Q // qb),
        in_specs=[
            pl.BlockSpec((1, qb, 8), lambda b, i: (b, i, 0)),
            pl.BlockSpec((1, N, 8), lambda b, i: (b, 0, 0)),
        ],
        out_specs=pl.BlockSpec((1, qb, Kp), lambda b, i: (b, i, 0)),
        out_shape=jax.ShapeDtypeStruct((Bc, Q, Kp), jnp.int32),
    )(query8, table8)


def _gather_rows(table2d, gidx):
    return jnp.take(table2d, gidx, axis=0)


def _pool_kernel(g_ref, cq_ref, w_ref, b_ref, mx_ref, av_ref, *, K, Kp):
    g = g_ref[0]
    cq = cq_ref[0]
    qb, _, c1 = g.shape
    h1 = jnp.maximum(g - cq[:, None, :], 0.0)
    h1f = h1.reshape(qb * Kp, c1)
    h2 = jnp.dot(h1f, w_ref[...], preferred_element_type=jnp.float32)
    h2 = jnp.maximum(h2 + b_ref[...], 0.0)
    c2 = h2.shape[1]
    h2 = h2.reshape(qb, Kp, c2)
    if Kp > K:
        kmask = (jax.lax.broadcasted_iota(jnp.int32, (1, Kp, 1), 1) <
                 K).astype(jnp.float32)
        h2 = h2 * kmask
    mx_ref[0] = jnp.max(h2, axis=1)
    av_ref[0] = jnp.sum(h2, axis=1) * (1.0 / K)


def _pool_mlp_pallas(G, CQp, W2, b2, K, qb=256):
    Bc, Q, Kp, C1 = G.shape
    C2 = W2.shape[1]
    qb = min(qb, Q)
    kern = functools.partial(_pool_kernel, K=K, Kp=Kp)
    mx, av = pl.pallas_call(
        kern,
        grid=(Bc, Q // qb),
        in_specs=[
            pl.BlockSpec((1, qb, Kp, C1), lambda b, i: (b, i, 0, 0)),
            pl.BlockSpec((1, qb, C1), lambda b, i: (b, i, 0)),
            pl.BlockSpec((C1, C2), lambda b, i: (0, 0)),
            pl.BlockSpec((1, C2), lambda b, i: (0, 0)),
        ],
        out_specs=[
            pl.BlockSpec((1, qb, C2), lambda b, i: (b, i, 0)),
            pl.BlockSpec((1, qb, C2), lambda b, i: (b, i, 0)),
        ],
        out_shape=[
            jax.ShapeDtypeStruct((Bc, Q, C2), jnp.float32),
            jax.ShapeDtypeStruct((Bc, Q, C2), jnp.float32),
        ],
    )(G, CQp, W2, b2.reshape(1, C2))
    return mx, av


def _fps(xyz, npoint):
    Bc, Nc, _ = xyz.shape
    def step(state, _):
        distance, farthest = state
        centroid = jax.vmap(lambda p, i: p[i])(xyz, farthest)[:, None, :]
        dist = jnp.sum((xyz - centroid) ** 2, -1)
        distance = jnp.minimum(distance, dist)
        new_far = jnp.argmax(distance, -1).astype(jnp.int32)
        return (distance, new_far), farthest
    init = (jnp.full((Bc, Nc), 1e10, dtype=jnp.float32),
            jnp.zeros((Bc,), jnp.int32))
    _, idxs = jax.lax.scan(step, init, None, length=npoint)
    return jnp.transpose(idxs)


def _neighbor_layer(radius, K, Kp, table8, query8, P, CQtab, W2, b2, qb=256):
    Bc, N, _ = table8.shape
    Q = query8.shape[1]
    C1 = P.shape[1]
    idxg = _ball_query_pallas(radius, K, Kp, table8, query8, qb=qb)
    G = _gather_rows(P, idxg.reshape(-1)).reshape(Bc, Q, Kp, C1)
    mx, av = _pool_mlp_pallas(G, CQtab, W2, b2, K, qb=qb)
    return mx, av


def _cos_loss(maxf, avgf):
    nm = jnp.sqrt(jnp.sum(maxf ** 2, -1)) * jnp.sqrt(jnp.sum(avgf ** 2, -1)) + 1e-8
    return jnp.mean(jnp.sum(maxf * avgf, -1) / nm)


def _batchnorm1d(x, g, b):
    m = jnp.mean(x, 0)
    v = jnp.var(x, 0)
    return (x - m) / jnp.sqrt(v + 1e-5) * g + b


def kernel(xyz, params):
    Bc = xyz.shape[0]
    N = xyz.shape[2]
    xyz_t = jnp.transpose(xyz, (0, 2, 1))
    x8 = _pad8(xyz_t)
    xyz_f = xyz_t.reshape(Bc * N, 3)

    reW, reb = params['re_W'], params['re_b']
    Wn, Wg = reW[0][0:3], reW[0][3:6]
    P_re = xyz_f @ (Wn + Wg)
    CQ_re = (xyz_f @ Wn - reb[0]).reshape(Bc, N, 32)
    mx, av = _neighbor_layer(0.2, 32, 32, x8, x8, P_re, CQ_re,
                             reW[1], reb[1])
    points = jnp.concatenate([mx, av], -1)

    saW, sab = params['sa1_W'], params['sa1_b']
    Wn, Wx, Wp = saW[0][0:3], saW[0][3:6], saW[0][6:]
    P1 = xyz_f @ (Wn + Wx) + points.reshape(Bc * N, 64) @ Wp
    CQ1tab = (xyz_f @ Wn - sab[0])
    fps1 = _fps(xyz_t, 512)
    g1 = (fps1 + jnp.arange(Bc, dtype=jnp.int32)[:, None] * N).reshape(-1)
    new_xyz1 = jnp.take(xyz_f, g1, axis=0).reshape(Bc, 512, 3)
    CQ1 = jnp.take(CQ1tab, g1, axis=0).reshape(Bc, 512, 64)
    mx, av = _neighbor_layer(0.2, 32, 32, x8, _pad8(new_xyz1), P1, CQ1,
                             saW[1], sab[1])
    c1 = _cos_loss(mx, av)
    l1_pts = jnp.concatenate([mx, av], -1)

    saW, sab = params['sa2_W'], params['sa2_b']
    Wn, Wx, Wp = saW[0][0:3], saW[0][3:6], saW[0][6:]
    l1_f = new_xyz1.reshape(Bc * 512, 3)
    P2 = l1_f @ (Wn + Wx) + l1_pts.reshape(Bc * 512, 128) @ Wp
    CQ2tab = (l1_f @ Wn - sab[0])
    fps2 = _fps(new_xyz1, 128)
    g2 = (fps2 + jnp.arange(Bc, dtype=jnp.int32)[:, None] * 512).reshape(-1)
    new_xyz2 = jnp.take(l1_f, g2, axis=0).reshape(Bc, 128, 3)
    CQ2 = jnp.take(CQ2tab, g2, axis=0).reshape(Bc, 128, 128)
    mx, av = _neighbor_layer(0.4, 20, 24, _pad8(new_xyz1), _pad8(new_xyz2),
                             P2, CQ2, saW[1], sab[1], qb=128)
    c2 = _cos_loss(mx, av)
    l2_pts = jnp.concatenate([mx, av], -1)

    W3, b3 = params['sa3_W'][0], params['sa3_b'][0]
    W3eff = jnp.concatenate([W3[0:3] + W3[3:6], W3[6:]], axis=0)
    feat3 = jnp.concatenate([new_xyz2, l2_pts], -1).reshape(Bc * 128, 515)
    h3 = _dense_relu(feat3, W3eff, b3).reshape(Bc, 128, 1024)
    x = jnp.max(h3, axis=1)

    x = jax.nn.leaky_relu(
        _batchnorm1d(x @ params['fc1_W'] + params['fc1_b'],
                     params['bn1_g'], params['bn1_b']), 0.2)
    x = jax.nn.leaky_relu(
        _batchnorm1d(x @ params['fc2_W'] + params['fc2_b'],
                     params['bn2_g'], params['bn2_b']), 0.2)
    x = x @ params['fc3_W'] + params['fc3_b']
    cos_loss = c1 + c2 + jnp.float32(0.0)
    return (x, cos_loss, jnp.asarray(512, jnp.int32), jnp.asarray(128, jnp.int32))


# speedup vs baseline: 3.5576x; 1.0197x over previous
import functools

import jax
import jax.numpy as jnp
from jax.experimental import pallas as pl
from jax.experimental.pallas import tpu as pltpu

B = 8


def _dense_relu_kernel(x_ref, w_ref, b_ref, o_ref):
    acc = jnp.dot(x_ref[...], w_ref[...], preferred_element_type=jnp.float32)
    o_ref[...] = jnp.maximum(acc + b_ref[...], 0.0)


def _dense_relu(x2d, W, b, block_rows=1024):
    R, Cin = x2d.shape
    Cout = W.shape[1]
    br = min(block_rows, max(8, R))
    Rp = ((R + br - 1) // br) * br
    if Rp != R:
        x2d = jnp.pad(x2d, ((0, Rp - R), (0, 0)))
    out = pl.pallas_call(
        _dense_relu_kernel,
        grid=(Rp // br,),
        in_specs=[
            pl.BlockSpec((br, Cin), lambda i: (i, 0)),
            pl.BlockSpec((Cin, Cout), lambda i: (0, 0)),
            pl.BlockSpec((1, Cout), lambda i: (0, 0)),
        ],
        out_specs=pl.BlockSpec((br, Cout), lambda i: (i, 0)),
        out_shape=jax.ShapeDtypeStruct((Rp, Cout), jnp.float32),
    )(x2d, W, b.reshape(1, Cout))
    return out[:R]


def _bq_kernel(q_ref, t_ref, qn_ref, tn_ref, o_ref, *, radius, K, Kp, N):
    q = q_ref[0]
    tT = t_ref[0]
    b = pl.program_id(0)
    dot = jax.lax.dot_general(q, tT, (((1,), (0,)), ((), ())),
                              preferred_element_type=jnp.float32)
    qn2 = qn_ref[0]
    tn2 = tn_ref[0]
    d = -2.0 * dot + qn2 + tn2
    mask = (d <= radius * radius).astype(jnp.float32)
    ch = min(128, N)
    row_i = jax.lax.broadcasted_iota(jnp.int32, (ch, ch), 0)
    col_i = jax.lax.broadcasted_iota(jnp.int32, (ch, ch), 1)
    ut = (row_i <= col_i).astype(jnp.float32)
    chunks = []
    carry = jnp.zeros((q.shape[0], 1), jnp.float32)
    for c in range(N // ch):
        blk = mask[:, c * ch:(c + 1) * ch]
        r = jnp.dot(blk, ut, preferred_element_type=jnp.float32) + carry
        carry = r[:, ch - 1:ch]
        chunks.append(r)
    rank = jnp.concatenate(chunks, axis=1)
    cols = []
    for k in range(K):
        cols.append(jnp.sum(jnp.where(rank <= float(k), 1.0, 0.0),
                            axis=-1, keepdims=True))
    idx = jnp.concatenate(cols, axis=1)
    first = idx[:, 0:1]
    idx = jnp.where(idx >= float(N), first, idx)
    if Kp > K:
        idx = jnp.concatenate(
            [idx, jnp.zeros((idx.shape[0], Kp - K), jnp.float32)], axis=1)
    o_ref[0] = idx.astype(jnp.int32) + b * N


def _pad8(x3):
    B_, M, _ = x3.shape
    return jnp.concatenate([x3, jnp.zeros((B_, M, 5), jnp.float32)], axis=-1)


def _ball_query_pallas(radius, K, Kp, table8, query8, qb=256):
    Bc, N, _ = table8.shape
    Q = query8.shape[1]
    qb = min(qb, Q)
    kern = functools.partial(_bq_kernel, radius=radius, K=K, Kp=Kp, N=N)
    qn2 = jnp.sum(query8[:, :, 0:3] ** 2, -1)[:, :, None]
    tn2 = jnp.sum(table8[:, :, 0:3] ** 2, -1)[:, None, :]
    return pl.pallas_call(
        kern,
        grid=(Bc, Q // qb),
        in_specs=[
            pl.BlockSpec((1, qb, 8), lambda b, i: (b, i, 0)),
            pl.BlockSpec((1, 8, N), lambda b, i: (b, 0, 0)),
            pl.BlockSpec((1, qb, 1), lambda b, i: (b, i, 0)),
            pl.BlockSpec((1, 1, N), lambda b, i: (b, 0, 0)),
        ],
        out_specs=pl.BlockSpec((1, qb, Kp), lambda b, i: (b, i, 0)),
        out_shape=jax.ShapeDtypeStruct((Bc, Q, Kp), jnp.int32),
    )(query8, jnp.transpose(table8, (0, 2, 1)), qn2, tn2)


def _gather_rows(table2d, gidx):
    return jnp.take(table2d, gidx, axis=0)


def _pool_kernel(g_ref, cq_ref, w1_ref, b1_ref, w2_ref, b2_ref,
                 mx_ref, av_ref, *, K, Kp, Cp):
    g = g_ref[0]
    cq = cq_ref[0]
    qb = g.shape[0]
    gx = g[:, :, 0:3]
    norm = gx - cq[:, None, 0:3]
    parts = [norm, gx]
    if Cp:
        parts.append(g[:, :, 8:8 + Cp])
    feat = jnp.concatenate(parts, axis=-1)
    cf = feat.shape[-1]
    h = feat.reshape(qb * Kp, cf)
    h = jnp.maximum(jnp.dot(h, w1_ref[...],
                            preferred_element_type=jnp.float32)
                    + b1_ref[...], 0.0)
    h = jnp.maximum(jnp.dot(h, w2_ref[...],
                            preferred_element_type=jnp.float32)
                    + b2_ref[...], 0.0)
    c2 = h.shape[1]
    h = h.reshape(qb, Kp, c2)
    if Kp > K:
        kmask = (jax.lax.broadcasted_iota(jnp.int32, (1, Kp, 1), 1) <
                 K).astype(jnp.float32)
        h = h * kmask
    mx_ref[0] = jnp.max(h, axis=1)
    av_ref[0] = jnp.sum(h, axis=1) / K


def _pool_mlp_pallas(G, Q8, W1, b1, W2, b2, K, qb=256):
    Bc, Q, Kp, Ct = G.shape
    Cp = Ct - 8
    C1 = W1.shape[1]
    C2 = W2.shape[1]
    qb = min(qb, Q)
    kern = functools.partial(_pool_kernel, K=K, Kp=Kp, Cp=Cp)
    mx, av = pl.pallas_call(
        kern,
        grid=(Bc, Q // qb),
        in_specs=[
            pl.BlockSpec((1, qb, Kp, Ct), lambda b, i: (b, i, 0, 0)),
            pl.BlockSpec((1, qb, 8), lambda b, i: (b, i, 0)),
            pl.BlockSpec(W1.shape, lambda b, i: (0, 0)),
            pl.BlockSpec((1, C1), lambda b, i: (0, 0)),
            pl.BlockSpec((C1, C2), lambda b, i: (0, 0)),
            pl.BlockSpec((1, C2), lambda b, i: (0, 0)),
        ],
        out_specs=[
            pl.BlockSpec((1, qb, C2), lambda b, i: (b, i, 0)),
            pl.BlockSpec((1, qb, C2), lambda b, i: (b, i, 0)),
        ],
        out_shape=[
            jax.ShapeDtypeStruct((Bc, Q, C2), jnp.float32),
            jax.ShapeDtypeStruct((Bc, Q, C2), jnp.float32),
        ],
    )(G, Q8, W1, b1.reshape(1, C1), W2, b2.reshape(1, C2))
    return mx, av


def _fps(xyz, npoint):
    Bc, Nc, _ = xyz.shape
    def step(state, _):
        distance, farthest = state
        centroid = jax.vmap(lambda p, i: p[i])(xyz, farthest)[:, None, :]
        dist = jnp.sum((xyz - centroid) ** 2, -1)
        distance = jnp.minimum(distance, dist)
        new_far = jnp.argmax(distance, -1).astype(jnp.int32)
        return (distance, new_far), farthest
    init = (jnp.full((Bc, Nc), 1e10, dtype=jnp.float32),
            jnp.zeros((Bc,), jnp.int32))
    _, idxs = jax.lax.scan(step, init, None, length=npoint)
    return jnp.transpose(idxs)


def _neighbor_layer(radius, K, Kp, table8, query8, feats, Ws, bs, qb=256):
    Bc, N, _ = table8.shape
    Q = query8.shape[1]
    if feats is not None:
        T = jnp.concatenate([table8, feats], axis=-1)
    else:
        T = table8
    Ct = T.shape[-1]
    idxg = _ball_query_pallas(radius, K, Kp, table8, query8, qb=qb)
    G = _gather_rows(T.reshape(Bc * N, Ct), idxg.reshape(-1))
    G = G.reshape(Bc, Q, Kp, Ct)
    return _pool_mlp_pallas(G, query8, Ws[0], bs[0], Ws[1], bs[1], K, qb=qb)


def _cos_loss(maxf, avgf):
    nm = jnp.sqrt(jnp.sum(maxf ** 2, -1)) * jnp.sqrt(jnp.sum(avgf ** 2, -1)) + 1e-8
    return jnp.mean(jnp.sum(maxf * avgf, -1) / nm)


def _batchnorm1d(x, g, b):
    m = jnp.mean(x, 0)
    v = jnp.var(x, 0)
    return (x - m) / jnp.sqrt(v + 1e-5) * g + b


def kernel(xyz, params):
    Bc = xyz.shape[0]
    N = xyz.shape[2]
    xyz_t = jnp.transpose(xyz, (0, 2, 1))
    x8 = _pad8(xyz_t)
    xyz_f = xyz_t.reshape(Bc * N, 3)

    mx, av = _neighbor_layer(0.2, 32, 32, x8, x8, None,
                             params['re_W'], params['re_b'])
    points = jnp.concatenate([mx, av], -1)

    fps1 = _fps(xyz_t, 512)
    g1 = (fps1 + jnp.arange(Bc, dtype=jnp.int32)[:, None] * N).reshape(-1)
    new_xyz1 = jnp.take(xyz_f, g1, axis=0).reshape(Bc, 512, 3)
    mx, av = _neighbor_layer(0.2, 32, 32, x8, _pad8(new_xyz1), points,
                             params['sa1_W'], params['sa1_b'])
    c1 = _cos_loss(mx, av)
    l1_pts = jnp.concatenate([mx, av], -1)

    l1_f = new_xyz1.reshape(Bc * 512, 3)
    fps2 = _fps(new_xyz1, 128)
    g2 = (fps2 + jnp.arange(Bc, dtype=jnp.int32)[:, None] * 512).reshape(-1)
    new_xyz2 = jnp.take(l1_f, g2, axis=0).reshape(Bc, 128, 3)
    mx, av = _neighbor_layer(0.4, 20, 24, _pad8(new_xyz1), _pad8(new_xyz2),
                             l1_pts, params['sa2_W'], params['sa2_b'], qb=128)
    c2 = _cos_loss(mx, av)
    l2_pts = jnp.concatenate([mx, av], -1)

    W3, b3 = params['sa3_W'][0], params['sa3_b'][0]
    feat3 = jnp.concatenate([new_xyz2, new_xyz2, l2_pts],
                            -1).reshape(Bc * 128, 518)
    h3 = _dense_relu(feat3, W3, b3).reshape(Bc, 128, 1024)
    x = jnp.max(h3, axis=1)

    x = jax.nn.leaky_relu(
        _batchnorm1d(x @ params['fc1_W'] + params['fc1_b'],
                     params['bn1_g'], params['bn1_b']), 0.2)
    x = jax.nn.leaky_relu(
        _batchnorm1d(x @ params['fc2_W'] + params['fc2_b'],
                     params['bn2_g'], params['bn2_b']), 0.2)
    x = x @ params['fc3_W'] + params['fc3_b']
    cos_loss = c1 + c2 + jnp.float32(0.0)
    return (x, cos_loss, jnp.asarray(512, jnp.int32), jnp.asarray(128, jnp.int32))
